# baseline (speedup 1.0000x reference)
import functools

import jax
import jax.numpy as jnp
from jax import lax
from jax.experimental import pallas as pl
from jax.experimental.pallas import tpu as pltpu
from jax.experimental.pallas import tpu_sc as plsc

N = 10000
D = 128
E = 320000
NP = 10240
NC = 2
NS = 16
L = 16
NW = NC * NS
EB = 128
NBE = 80
EPT = NBE * EB
EPAD = NW * EPT
NBUF = 4
RPT = NP // NW
RC = 80
SEG = NP // NS

_MESH = plsc.VectorSubcoreMesh(core_axis_name="c", subcore_axis_name="s")


def _wid():
    return lax.axis_index("s") * NC + lax.axis_index("c")


def _hist_body(dst_hbm, out_hbm, idx_v, ones_v, zer_v, hist_sh, sem):
    c = lax.axis_index("c")
    s = lax.axis_index("s")

    def fill_zero(i, _):
        zer_v[pl.ds(i * L, L)] = jnp.zeros((L,), jnp.float32)
        return 0

    lax.fori_loop(0, SEG // L, fill_zero, 0)
    for i in range(EB // L):
        ones_v[pl.ds(i * L, L)] = jnp.ones((L,), jnp.float32)
    pltpu.sync_copy(zer_v, hist_sh.at[pl.ds(s * SEG, SEG)])
    plsc.subcore_barrier()

    pltpu.sync_copy(dst_hbm.at[_wid()], idx_v)

    def step(j, _):
        pltpu.sync_copy(ones_v, hist_sh.at[idx_v.at[j]], add=True)
        return 0

    lax.fori_loop(0, NBE, step, 0)
    plsc.subcore_barrier()
    pltpu.sync_copy(hist_sh.at[pl.ds(s * SEG, SEG)],
                    out_hbm.at[c, pl.ds(s * SEG, SEG)])


_hist = functools.partial(
    pl.kernel,
    out_type=jax.ShapeDtypeStruct((NC, NP), jnp.float32),
    mesh=_MESH,
    scratch_types=[
        pltpu.VMEM((NBE, EB), jnp.int32),
        pltpu.VMEM((EB,), jnp.float32),
        pltpu.VMEM((SEG,), jnp.float32),
        pltpu.VMEM_SHARED((NP,), jnp.float32),
        pltpu.SemaphoreType.DMA,
    ],
)(_hist_body)


DH = D // 2


def _agg_body(ya_hbm, yb_hbm, src_hbm, dst_hbm, out_hbm, si_v, di_v,
              *scratch):
    rows = scratch[:NBUF]
    zr_v = scratch[NBUF]
    acc_sh = scratch[NBUF + 1]
    gsem = scratch[NBUF + 2:NBUF + 2 + NBUF]
    ssem = scratch[NBUF + 2 + NBUF:]
    c = lax.axis_index("c")
    s = lax.axis_index("s")

    pltpu.sync_copy(src_hbm.at[_wid()], si_v)
    pltpu.sync_copy(dst_hbm.at[_wid()], di_v)

    def zero_row(i, _):
        for cc in range(DH // L):
            zr_v[i, pl.ds(cc * L, L)] = jnp.zeros((L,), jnp.float32)
        return 0

    lax.fori_loop(0, EB, zero_row, 0)

    for h, y_hbm in enumerate((ya_hbm, yb_hbm)):
        @pl.when(c == 0)
        def _():
            pltpu.sync_copy(y_hbm.at[pl.ds(s * SEG, SEG)],
                            acc_sh.at[pl.ds(s * SEG, SEG)])

        @pl.when(c != 0)
        def _():
            for k in range(SEG // EB):
                pltpu.sync_copy(zr_v, acc_sh.at[pl.ds(s * SEG + k * EB, EB)])

        plsc.subcore_barrier()

        for b in range(NBUF):
            pltpu.async_copy(y_hbm.at[si_v.at[b]], rows[b], gsem[b])

        def group(g, _):
            base = g * NBUF
            for b in range(NBUF):
                j = base + b
                pltpu.make_async_copy(y_hbm.at[si_v.at[j]], rows[b],
                                      gsem[b]).wait()
                pltpu.async_copy(rows[b], acc_sh.at[di_v.at[j]], ssem[b],
                                 add=True)
            for b in range(NBUF):
                j = base + b
                pltpu.make_async_copy(rows[b], acc_sh.at[di_v.at[j]],
                                      ssem[b]).wait()
                pltpu.async_copy(y_hbm.at[si_v.at[j + NBUF]], rows[b],
                                 gsem[b])
            return 0

        lax.fori_loop(0, NBE // NBUF - 1, group, 0)
        for b in range(NBUF):
            j = NBE - NBUF + b
            pltpu.make_async_copy(y_hbm.at[si_v.at[j]], rows[b],
                                  gsem[b]).wait()
            pltpu.async_copy(rows[b], acc_sh.at[di_v.at[j]], ssem[b],
                             add=True)
        for b in range(NBUF):
            j = NBE - NBUF + b
            pltpu.make_async_copy(rows[b], acc_sh.at[di_v.at[j]],
                                  ssem[b]).wait()
        plsc.subcore_barrier()
        pltpu.sync_copy(acc_sh.at[pl.ds(s * SEG, SEG)],
                        out_hbm.at[c, h, pl.ds(s * SEG, SEG)])


_agg = functools.partial(
    pl.kernel,
    out_type=jax.ShapeDtypeStruct((NC, 2, NP, DH), jnp.float32),
    mesh=_MESH,
    compiler_params=pltpu.CompilerParams(use_tc_tiling_on_sc=False),
    scratch_types=(
        [
            pltpu.VMEM((NBE, EB), jnp.int32),
            pltpu.VMEM((NBE, EB), jnp.int32),
        ]
        + [pltpu.VMEM((EB, DH), jnp.float32) for _ in range(NBUF)]
        + [
            pltpu.VMEM((EB, DH), jnp.float32),
            pltpu.VMEM_SHARED((NP, DH), jnp.float32),
        ]
        + [pltpu.SemaphoreType.DMA for _ in range(2 * NBUF)]
    ),
)(_agg_body)


_PREC = lax.Precision.HIGHEST
MB = 128


def _diag(h_ref):
    deg = h_ref[0, 0:1] + h_ref[0, 1:2] + 1.0
    dv = lax.rsqrt(deg)
    ior = lax.broadcasted_iota(jnp.int32, (MB, MB), 0)
    ioc = lax.broadcasted_iota(jnp.int32, (MB, MB), 1)
    return jnp.where(ior == ioc, jnp.broadcast_to(dv, (MB, MB)), 0.0)


def _dot(a, b):
    return lax.dot_general(a, b, (((1,), (0,)), ((), ())),
                           preferred_element_type=jnp.float32,
                           precision=_PREC)


def _dot_t(a, b):
    return lax.dot_general(a, b, (((1,), (1,)), ((), ())),
                           preferred_element_type=jnp.float32,
                           precision=_PREC)


def _hist_t(hist):
    return hist.reshape(NC, NP // MB, MB).transpose(1, 0, 2)

def _k1_body(h_ref, x_ref, w_ref, o_ref):
    o_ref[...] = _dot_t(_dot(_diag(h_ref), x_ref[...]), w_ref[...])


def _k1(hist, x_pad, w1):
    return pl.pallas_call(
        _k1_body,
        grid=(NP // MB,),
        in_specs=[
            pl.BlockSpec((1, NC, MB), lambda i: (i, 0, 0)),
            pl.BlockSpec((MB, D), lambda i: (i, 0)),
            pl.BlockSpec((D, D), lambda i: (0, 0)),
        ],
        out_specs=pl.BlockSpec((MB, D), lambda i: (i, 0)),
        out_shape=jax.ShapeDtypeStruct((NP, D), jnp.float32),
    )(_hist_t(hist), x_pad, w1)


def _k2_body(h_ref, pa0, pa1, pb0, pb1, b_ref, w_ref, o_ref):
    dg = _diag(h_ref)
    tl = _dot(dg, pa0[...] + pa1[...]) + b_ref[:, :DH]
    tr = _dot(dg, pb0[...] + pb1[...]) + b_ref[:, DH:]
    hl = _dot(dg, jnp.maximum(tl, 0.0))
    hr = _dot(dg, jnp.maximum(tr, 0.0))
    o_ref[...] = _dot_t(hl, w_ref[:, :DH]) + _dot_t(hr, w_ref[:, DH:])


def _k2(hist, p, b1, w2):
    return pl.pallas_call(
        _k2_body,
        grid=(NP // MB,),
        in_specs=[
            pl.BlockSpec((1, NC, MB), lambda i: (i, 0, 0)),
            pl.BlockSpec((MB, DH), lambda i: (i, 0)),
            pl.BlockSpec((MB, DH), lambda i: (i, 0)),
            pl.BlockSpec((MB, DH), lambda i: (i, 0)),
            pl.BlockSpec((MB, DH), lambda i: (i, 0)),
            pl.BlockSpec((1, D), lambda i: (0, 0)),
            pl.BlockSpec((D, D), lambda i: (0, 0)),
        ],
        out_specs=pl.BlockSpec((MB, D), lambda i: (i, 0)),
        out_shape=jax.ShapeDtypeStruct((NP, D), jnp.float32),
    )(_hist_t(hist), p[0, 0], p[1, 0], p[0, 1], p[1, 1],
      b1.reshape(1, D), w2)


def _k3_body(h_ref, pa0, pa1, pb0, pb1, b_ref, o_ref):
    dg = _diag(h_ref)
    o_ref[:, :DH] = _dot(dg, pa0[...] + pa1[...]) + b_ref[:, :DH]
    o_ref[:, DH:] = _dot(dg, pb0[...] + pb1[...]) + b_ref[:, DH:]


def _k3(hist, p, b2):
    return pl.pallas_call(
        _k3_body,
        grid=(NP // MB,),
        in_specs=[
            pl.BlockSpec((1, NC, MB), lambda i: (i, 0, 0)),
            pl.BlockSpec((MB, DH), lambda i: (i, 0)),
            pl.BlockSpec((MB, DH), lambda i: (i, 0)),
            pl.BlockSpec((MB, DH), lambda i: (i, 0)),
            pl.BlockSpec((MB, DH), lambda i: (i, 0)),
            pl.BlockSpec((1, D), lambda i: (0, 0)),
        ],
        out_specs=pl.BlockSpec((MB, D), lambda i: (i, 0)),
        out_shape=jax.ShapeDtypeStruct((NP, D), jnp.float32),
    )(_hist_t(hist), p[0, 0], p[1, 0], p[0, 1], p[1, 1],
      b2.reshape(1, D))


def kernel(x, edge_index, W1, b1, W2, b2):
    src = edge_index[0].astype(jnp.int32)
    dst = edge_index[1].astype(jnp.int32)
    pad = N + jnp.arange(EPAD - E, dtype=jnp.int32) % (NP - N)
    src3 = jnp.concatenate([src, pad]).reshape(NW, NBE, EB)
    dst3 = jnp.concatenate([dst, pad]).reshape(NW, NBE, EB)
    x_pad = jnp.zeros((NP, D), jnp.float32).at[:N].set(x)

    hist = _hist(dst3)
    y1 = _k1(hist, x_pad, W1)
    p1 = _agg(y1[:, :DH], y1[:, DH:], src3, dst3)
    y2 = _k2(hist, p1, b1, W2)
    p2 = _agg(y2[:, :DH], y2[:, DH:], src3, dst3)
    out = _k3(hist, p2, b2)
    return out[:N]


# speedup vs baseline: 1.2526x; 1.2526x over previous
import functools

import jax
import jax.numpy as jnp
from jax import lax
from jax.experimental import pallas as pl
from jax.experimental.pallas import tpu as pltpu
from jax.experimental.pallas import tpu_sc as plsc

N = 10000
D = 128
E = 320000
NP = 10240
NC = 2
NS = 16
L = 16
NW = NC * NS
EB = 128
NBE = 80
EPT = NBE * EB
EPAD = NW * EPT
NBUF = 4
RPT = NP // NW
RC = 80
SEG = NP // NS

_MESH = plsc.VectorSubcoreMesh(core_axis_name="c", subcore_axis_name="s")


def _wid():
    return lax.axis_index("s") * NC + lax.axis_index("c")


def _hist_body(dst_hbm, out_hbm, idx_v, ones_v, zer_v, hist_sh, sem):
    c = lax.axis_index("c")
    s = lax.axis_index("s")

    def fill_zero(i, _):
        zer_v[pl.ds(i * L, L)] = jnp.zeros((L,), jnp.float32)
        return 0

    lax.fori_loop(0, SEG // L, fill_zero, 0)
    for i in range(EB // L):
        ones_v[pl.ds(i * L, L)] = jnp.ones((L,), jnp.float32)
    pltpu.sync_copy(zer_v, hist_sh.at[pl.ds(s * SEG, SEG)])
    plsc.subcore_barrier()

    pltpu.sync_copy(dst_hbm.at[_wid()], idx_v)

    def step(j, _):
        pltpu.sync_copy(ones_v, hist_sh.at[idx_v.at[j]], add=True)
        return 0

    lax.fori_loop(0, NBE, step, 0)
    plsc.subcore_barrier()
    pltpu.sync_copy(hist_sh.at[pl.ds(s * SEG, SEG)],
                    out_hbm.at[c, pl.ds(s * SEG, SEG)])


_hist = functools.partial(
    pl.kernel,
    out_type=jax.ShapeDtypeStruct((NC, NP), jnp.float32),
    mesh=_MESH,
    scratch_types=[
        pltpu.VMEM((NBE, EB), jnp.int32),
        pltpu.VMEM((EB,), jnp.float32),
        pltpu.VMEM((SEG,), jnp.float32),
        pltpu.VMEM_SHARED((NP,), jnp.float32),
        pltpu.SemaphoreType.DMA,
    ],
)(_hist_body)


DH = D // 2


def _agg_body(ya_hbm, yb_hbm, src_hbm, dst_hbm, out_hbm, si_v, di_v,
              *scratch):
    rows = scratch[:NBUF]
    zr_v = scratch[NBUF]
    acc_sh = scratch[NBUF + 1]
    gsem = scratch[NBUF + 2:NBUF + 2 + NBUF]
    ssem = scratch[NBUF + 2 + NBUF:]
    c = lax.axis_index("c")
    s = lax.axis_index("s")

    pltpu.sync_copy(src_hbm.at[_wid()], si_v)
    pltpu.sync_copy(dst_hbm.at[_wid()], di_v)

    def zero_row(i, _):
        for cc in range(DH // L):
            zr_v[i, pl.ds(cc * L, L)] = jnp.zeros((L,), jnp.float32)
        return 0

    lax.fori_loop(0, EB, zero_row, 0)

    for h, y_hbm in enumerate((ya_hbm, yb_hbm)):
        @pl.when(c == 0)
        def _():
            pltpu.sync_copy(y_hbm.at[pl.ds(s * SEG, SEG)],
                            acc_sh.at[pl.ds(s * SEG, SEG)])

        @pl.when(c != 0)
        def _():
            for k in range(SEG // EB):
                pltpu.sync_copy(zr_v, acc_sh.at[pl.ds(s * SEG + k * EB, EB)])

        plsc.subcore_barrier()

        for b in range(NBUF):
            pltpu.async_copy(y_hbm.at[si_v.at[b]], rows[b], gsem[b])

        def group(g, _):
            base = g * NBUF
            for b in range(NBUF):
                j = base + b
                pltpu.make_async_copy(y_hbm.at[si_v.at[j]], rows[b],
                                      gsem[b]).wait()
                pltpu.async_copy(rows[b], acc_sh.at[di_v.at[j]], ssem[b],
                                 add=True)
            for b in range(NBUF):
                j = base + b
                pltpu.make_async_copy(rows[b], acc_sh.at[di_v.at[j]],
                                      ssem[b]).wait()
                pltpu.async_copy(y_hbm.at[si_v.at[j + NBUF]], rows[b],
                                 gsem[b])
            return 0

        lax.fori_loop(0, NBE // NBUF - 1, group, 0)
        for b in range(NBUF):
            j = NBE - NBUF + b
            pltpu.make_async_copy(y_hbm.at[si_v.at[j]], rows[b],
                                  gsem[b]).wait()
            pltpu.async_copy(rows[b], acc_sh.at[di_v.at[j]], ssem[b],
                             add=True)
        for b in range(NBUF):
            j = NBE - NBUF + b
            pltpu.make_async_copy(rows[b], acc_sh.at[di_v.at[j]],
                                  ssem[b]).wait()
        plsc.subcore_barrier()
        pltpu.sync_copy(acc_sh.at[pl.ds(s * SEG, SEG)],
                        out_hbm.at[c, h, pl.ds(s * SEG, SEG)])


_agg = functools.partial(
    pl.kernel,
    out_type=jax.ShapeDtypeStruct((NC, 2, NP, DH), jnp.float32),
    mesh=_MESH,
    compiler_params=pltpu.CompilerParams(use_tc_tiling_on_sc=False),
    scratch_types=(
        [
            pltpu.VMEM((NBE, EB), jnp.int32),
            pltpu.VMEM((NBE, EB), jnp.int32),
        ]
        + [pltpu.VMEM((EB, DH), jnp.float32) for _ in range(NBUF)]
        + [
            pltpu.VMEM((EB, DH), jnp.float32),
            pltpu.VMEM_SHARED((NP, DH), jnp.float32),
        ]
        + [pltpu.SemaphoreType.DMA for _ in range(2 * NBUF)]
    ),
)(_agg_body)


_PREC = lax.Precision.HIGHEST
MB = 256


def _dot_t(a, b):
    return lax.dot_general(a, b, (((1,), (1,)), ((), ())),
                           preferred_element_type=jnp.float32,
                           precision=_PREC)


def _mm_body(x_ref, w_ref, o_ref):
    o_ref[...] = _dot_t(x_ref[...], w_ref[...])


def _mm(x, w):
    return pl.pallas_call(
        _mm_body,
        grid=(NP // MB,),
        in_specs=[
            pl.BlockSpec((MB, D), lambda i: (i, 0)),
            pl.BlockSpec((D, D), lambda i: (0, 0)),
        ],
        out_specs=pl.BlockSpec((MB, D), lambda i: (i, 0)),
        out_shape=jax.ShapeDtypeStruct((NP, D), jnp.float32),
    )(x, w)


def _dinv_body(h_ref, o_ref):
    o_ref[...] = lax.rsqrt(h_ref[0] + h_ref[1] + 1.0)


def _dinv(hist):
    return pl.pallas_call(
        _dinv_body,
        out_shape=jax.ShapeDtypeStruct((NP // D, D), jnp.float32),
    )(hist.reshape(NC, NP // D, D))


_P_SPECS = [
    pl.BlockSpec((1, 1, MB, DH), lambda i: (0, 0, i, 0)),
    pl.BlockSpec((1, 1, MB, DH), lambda i: (1, 0, i, 0)),
    pl.BlockSpec((1, 1, MB, DH), lambda i: (0, 1, i, 0)),
    pl.BlockSpec((1, 1, MB, DH), lambda i: (1, 1, i, 0)),
]
_H_SPEC = pl.BlockSpec((MB, DH), lambda i: (i, 0))


def _scale_body(dv_ref, x_ref, oa_ref, ob_ref):
    oa_ref[...] = dv_ref[:, :DH] * x_ref[:, :DH]
    ob_ref[...] = dv_ref[:, DH:] * x_ref[:, DH:]


def _scale(dvb, xw):
    return pl.pallas_call(
        _scale_body,
        grid=(NP // MB,),
        in_specs=[
            pl.BlockSpec((MB, D), lambda i: (i, 0)),
            pl.BlockSpec((MB, D), lambda i: (i, 0)),
        ],
        out_specs=[_H_SPEC, _H_SPEC],
        out_shape=[jax.ShapeDtypeStruct((NP, DH), jnp.float32)] * 2,
    )(dvb, xw)


def _k2_body(dv_ref, pa0, pa1, pb0, pb1, b_ref, w_ref, oa_ref, ob_ref):
    dl, dr = dv_ref[:, :DH], dv_ref[:, DH:]
    tl = dl * (pa0[0, 0] + pa1[0, 0]) + b_ref[:, :DH]
    tr = dr * (pb0[0, 0] + pb1[0, 0]) + b_ref[:, DH:]
    hl = dl * jnp.maximum(tl, 0.0)
    hr = dr * jnp.maximum(tr, 0.0)
    y2 = _dot_t(hl, w_ref[:, :DH]) + _dot_t(hr, w_ref[:, DH:])
    oa_ref[...] = y2[:, :DH]
    ob_ref[...] = y2[:, DH:]


def _k2(dvb, p, b1, w2):
    return pl.pallas_call(
        _k2_body,
        grid=(NP // MB,),
        in_specs=[pl.BlockSpec((MB, D), lambda i: (i, 0))] + _P_SPECS + [
            pl.BlockSpec((1, D), lambda i: (0, 0)),
            pl.BlockSpec((D, D), lambda i: (0, 0)),
        ],
        out_specs=[_H_SPEC, _H_SPEC],
        out_shape=[jax.ShapeDtypeStruct((NP, DH), jnp.float32)] * 2,
    )(dvb, p, p, p, p, b1.reshape(1, D), w2)


def _k3_body(dv_ref, pa0, pa1, pb0, pb1, b_ref, o_ref):
    o_ref[:, :DH] = dv_ref[:, :DH] * (pa0[0, 0] + pa1[0, 0]) + b_ref[:, :DH]
    o_ref[:, DH:] = dv_ref[:, DH:] * (pb0[0, 0] + pb1[0, 0]) + b_ref[:, DH:]


def _k3(dvb, p, b2):
    return pl.pallas_call(
        _k3_body,
        grid=(NP // MB,),
        in_specs=[pl.BlockSpec((MB, D), lambda i: (i, 0))] + _P_SPECS + [
            pl.BlockSpec((1, D), lambda i: (0, 0)),
        ],
        out_specs=pl.BlockSpec((MB, D), lambda i: (i, 0)),
        out_shape=jax.ShapeDtypeStruct((NP, D), jnp.float32),
    )(dvb, p, p, p, p, b2.reshape(1, D))


def kernel(x, edge_index, W1, b1, W2, b2):
    src = edge_index[0].astype(jnp.int32)
    dst = edge_index[1].astype(jnp.int32)
    pad = N + jnp.arange(EPAD - E, dtype=jnp.int32) % (NP - N)
    src3 = jnp.concatenate([src, pad]).reshape(NW, NBE, EB)
    dst3 = jnp.concatenate([dst, pad]).reshape(NW, NBE, EB)
    x_pad = jnp.zeros((NP, D), jnp.float32).at[:N].set(x)

    hist = _hist(dst3)
    xw1 = _mm(x_pad, W1)
    dvb = jnp.broadcast_to(
        _dinv(hist).reshape(NP)[:, None], (NP, D))
    ya, yb = _scale(dvb, xw1)
    p1 = _agg(ya, yb, src3, dst3)
    y2a, y2b = _k2(dvb, p1, b1, W2)
    p2 = _agg(y2a, y2b, src3, dst3)
    out = _k3(dvb, p2, b2)
    return out[:N]


# speedup vs baseline: 1.2897x; 1.0296x over previous
import functools

import numpy as _np

import jax
import jax.numpy as jnp
from jax import lax
from jax.experimental import pallas as pl
from jax.experimental.pallas import tpu as pltpu
from jax.experimental.pallas import tpu_sc as plsc

N = 10000
D = 128
E = 320000
NP = 10240
NC = 2
NS = 16
L = 16
NW = NC * NS
EB = 128
NBE = 80
EPT = NBE * EB
EPAD = NW * EPT
NBUF = 4
RPT = NP // NW
RC = 80
SEG = NP // NS

_MESH = plsc.VectorSubcoreMesh(core_axis_name="c", subcore_axis_name="s")


def _wid():
    return lax.axis_index("s") * NC + lax.axis_index("c")


def _hist_body(dst_hbm, out_hbm, idx_v, ones_v, zer_v, hist_sh, sem):
    c = lax.axis_index("c")
    s = lax.axis_index("s")

    def fill_zero(i, _):
        zer_v[pl.ds(i * L, L)] = jnp.zeros((L,), jnp.float32)
        return 0

    lax.fori_loop(0, SEG // L, fill_zero, 0)
    for i in range(EB // L):
        ones_v[pl.ds(i * L, L)] = jnp.ones((L,), jnp.float32)
    pltpu.sync_copy(zer_v, hist_sh.at[pl.ds(s * SEG, SEG)])
    plsc.subcore_barrier()

    pltpu.sync_copy(dst_hbm.at[_wid()], idx_v)

    def step(j, _):
        pltpu.sync_copy(ones_v, hist_sh.at[idx_v.at[j]], add=True)
        return 0

    lax.fori_loop(0, NBE, step, 0)
    plsc.subcore_barrier()
    pltpu.sync_copy(hist_sh.at[pl.ds(s * SEG, SEG)],
                    out_hbm.at[c, pl.ds(s * SEG, SEG)])


_hist = functools.partial(
    pl.kernel,
    out_type=jax.ShapeDtypeStruct((NC, NP), jnp.float32),
    mesh=_MESH,
    scratch_types=[
        pltpu.VMEM((NBE, EB), jnp.int32),
        pltpu.VMEM((EB,), jnp.float32),
        pltpu.VMEM((SEG,), jnp.float32),
        pltpu.VMEM_SHARED((NP,), jnp.float32),
        pltpu.SemaphoreType.DMA,
    ],
)(_hist_body)


DH = D // 2


def _agg_body(ya_hbm, yb_hbm, src_hbm, dst_hbm, out_hbm, si_v, di_v,
              *scratch):
    rows = scratch[:NBUF]
    zr_v = scratch[NBUF]
    acc_sh = scratch[NBUF + 1]
    gsem = scratch[NBUF + 2:NBUF + 2 + NBUF]
    ssem = scratch[NBUF + 2 + NBUF:]
    c = lax.axis_index("c")
    s = lax.axis_index("s")

    pltpu.sync_copy(src_hbm.at[_wid()], si_v)
    pltpu.sync_copy(dst_hbm.at[_wid()], di_v)

    def zero_row(i, _):
        for cc in range(DH // L):
            zr_v[i, pl.ds(cc * L, L)] = jnp.zeros((L,), jnp.float32)
        return 0

    lax.fori_loop(0, EB, zero_row, 0)

    for h, y_hbm in enumerate((ya_hbm, yb_hbm)):
        @pl.when(c == 0)
        def _():
            pltpu.sync_copy(y_hbm.at[pl.ds(s * SEG, SEG)],
                            acc_sh.at[pl.ds(s * SEG, SEG)])

        @pl.when(c != 0)
        def _():
            for k in range(SEG // EB):
                pltpu.sync_copy(zr_v, acc_sh.at[pl.ds(s * SEG + k * EB, EB)])

        plsc.subcore_barrier()

        for b in range(NBUF):
            pltpu.async_copy(y_hbm.at[si_v.at[b]], rows[b], gsem[b])

        def group(g, _):
            base = g * NBUF
            for b in range(NBUF):
                j = base + b
                pltpu.make_async_copy(y_hbm.at[si_v.at[j]], rows[b],
                                      gsem[b]).wait()
                pltpu.async_copy(rows[b], acc_sh.at[di_v.at[j]], ssem[b],
                                 add=True)
            for b in range(NBUF):
                j = base + b
                pltpu.make_async_copy(rows[b], acc_sh.at[di_v.at[j]],
                                      ssem[b]).wait()
                pltpu.async_copy(y_hbm.at[si_v.at[j + NBUF]], rows[b],
                                 gsem[b])
            return 0

        lax.fori_loop(0, NBE // NBUF - 1, group, 0)
        for b in range(NBUF):
            j = NBE - NBUF + b
            pltpu.make_async_copy(y_hbm.at[si_v.at[j]], rows[b],
                                  gsem[b]).wait()
            pltpu.async_copy(rows[b], acc_sh.at[di_v.at[j]], ssem[b],
                             add=True)
        for b in range(NBUF):
            j = NBE - NBUF + b
            pltpu.make_async_copy(rows[b], acc_sh.at[di_v.at[j]],
                                  ssem[b]).wait()
        plsc.subcore_barrier()
        pltpu.sync_copy(acc_sh.at[pl.ds(s * SEG, SEG)],
                        out_hbm.at[c, h, pl.ds(s * SEG, SEG)])


_agg = functools.partial(
    pl.kernel,
    out_type=jax.ShapeDtypeStruct((NC, 2, NP, DH), jnp.float32),
    mesh=_MESH,
    compiler_params=pltpu.CompilerParams(use_tc_tiling_on_sc=False),
    scratch_types=(
        [
            pltpu.VMEM((NBE, EB), jnp.int32),
            pltpu.VMEM((NBE, EB), jnp.int32),
        ]
        + [pltpu.VMEM((EB, DH), jnp.float32) for _ in range(NBUF)]
        + [
            pltpu.VMEM((EB, DH), jnp.float32),
            pltpu.VMEM_SHARED((NP, DH), jnp.float32),
        ]
        + [pltpu.SemaphoreType.DMA for _ in range(2 * NBUF)]
    ),
)(_agg_body)


_PREC = lax.Precision.HIGHEST
MB = 256


def _dot_t(a, b):
    return lax.dot_general(a, b, (((1,), (1,)), ((), ())),
                           preferred_element_type=jnp.float32,
                           precision=_PREC)


def _mm_body(x_ref, w_ref, o_ref):
    o_ref[...] = _dot_t(x_ref[...], w_ref[...])


def _mm(x, w):
    return pl.pallas_call(
        _mm_body,
        grid=(NP // MB,),
        in_specs=[
            pl.BlockSpec((MB, D), lambda i: (i, 0)),
            pl.BlockSpec((D, D), lambda i: (0, 0)),
        ],
        out_specs=pl.BlockSpec((MB, D), lambda i: (i, 0)),
        out_shape=jax.ShapeDtypeStruct((NP, D), jnp.float32),
    )(x, w)


def _dinv_body(h_ref, o_ref):
    o_ref[...] = lax.rsqrt(h_ref[0] + h_ref[1] + 1.0)


def _dinv(hist):
    return pl.pallas_call(
        _dinv_body,
        out_shape=jax.ShapeDtypeStruct((NP // D, D), jnp.float32),
    )(hist.reshape(NC, NP // D, D))


_P_SPECS = [
    pl.BlockSpec((1, 1, MB, DH), lambda i: (0, 0, i, 0)),
    pl.BlockSpec((1, 1, MB, DH), lambda i: (1, 0, i, 0)),
    pl.BlockSpec((1, 1, MB, DH), lambda i: (0, 1, i, 0)),
    pl.BlockSpec((1, 1, MB, DH), lambda i: (1, 1, i, 0)),
]
_H_SPEC = pl.BlockSpec((MB, DH), lambda i: (i, 0))


def _k1_body(dv_ref, x_ref, w_ref, oa_ref, ob_ref):
    y1 = _dot_t(dv_ref[...] * x_ref[...], w_ref[...])
    oa_ref[...] = y1[:, :DH]
    ob_ref[...] = y1[:, DH:]


def _k1(dvb, x, w):
    return pl.pallas_call(
        _k1_body,
        grid=(NP // MB,),
        in_specs=[
            pl.BlockSpec((MB, D), lambda i: (i, 0)),
            pl.BlockSpec((MB, D), lambda i: (i, 0)),
            pl.BlockSpec((D, D), lambda i: (0, 0)),
        ],
        out_specs=[_H_SPEC, _H_SPEC],
        out_shape=[jax.ShapeDtypeStruct((NP, DH), jnp.float32)] * 2,
    )(dvb, x, w)


def _k2_body(dv_ref, pa0, pa1, pb0, pb1, b_ref, w_ref, oa_ref, ob_ref):
    dl, dr = dv_ref[:, :DH], dv_ref[:, DH:]
    tl = dl * (pa0[0, 0] + pa1[0, 0]) + b_ref[:, :DH]
    tr = dr * (pb0[0, 0] + pb1[0, 0]) + b_ref[:, DH:]
    hl = dl * jnp.maximum(tl, 0.0)
    hr = dr * jnp.maximum(tr, 0.0)
    y2 = _dot_t(hl, w_ref[:, :DH]) + _dot_t(hr, w_ref[:, DH:])
    oa_ref[...] = y2[:, :DH]
    ob_ref[...] = y2[:, DH:]


def _k2(dvb, p, b1, w2):
    return pl.pallas_call(
        _k2_body,
        grid=(NP // MB,),
        in_specs=[pl.BlockSpec((MB, D), lambda i: (i, 0))] + _P_SPECS + [
            pl.BlockSpec((1, D), lambda i: (0, 0)),
            pl.BlockSpec((D, D), lambda i: (0, 0)),
        ],
        out_specs=[_H_SPEC, _H_SPEC],
        out_shape=[jax.ShapeDtypeStruct((NP, DH), jnp.float32)] * 2,
    )(dvb, p, p, p, p, b1.reshape(1, D), w2)


def _k3_body(dv_ref, pa0, pa1, pb0, pb1, b_ref, o_ref):
    o_ref[:, :DH] = dv_ref[:, :DH] * (pa0[0, 0] + pa1[0, 0]) + b_ref[:, :DH]
    o_ref[:, DH:] = dv_ref[:, DH:] * (pb0[0, 0] + pb1[0, 0]) + b_ref[:, DH:]


def _k3(dvb, p, b2):
    return pl.pallas_call(
        _k3_body,
        grid=(NP // MB,),
        in_specs=[pl.BlockSpec((MB, D), lambda i: (i, 0))] + _P_SPECS + [
            pl.BlockSpec((1, D), lambda i: (0, 0)),
        ],
        out_specs=pl.BlockSpec((MB, D), lambda i: (i, 0)),
        out_shape=jax.ShapeDtypeStruct((NP, D), jnp.float32),
    )(dvb, p, p, p, p, b2.reshape(1, D))


_PAD_IDX = _np.int32(N) + _np.arange(EPAD - E, dtype=_np.int32) % (NP - N)


def kernel(x, edge_index, W1, b1, W2, b2):
    src = edge_index[0].astype(jnp.int32)
    dst = edge_index[1].astype(jnp.int32)
    pad = jnp.asarray(_PAD_IDX)
    src3 = jnp.concatenate([src, pad]).reshape(NW, NBE, EB)
    dst3 = jnp.concatenate([dst, pad]).reshape(NW, NBE, EB)
    x_pad = jnp.zeros((NP, D), jnp.float32).at[:N].set(x)

    hist = _hist(dst3)
    dvb = jnp.broadcast_to(
        _dinv(hist).reshape(NP)[:, None], (NP, D))
    ya, yb = _k1(dvb, x_pad, W1)
    p1 = _agg(ya, yb, src3, dst3)
    y2a, y2b = _k2(dvb, p1, b1, W2)
    p2 = _agg(y2a, y2b, src3, dst3)
    out = _k3(dvb, p2, b2)
    return out[:N]


# speedup vs baseline: 1.4045x; 1.0890x over previous
import functools

import numpy as _np

import jax
import jax.numpy as jnp
from jax import lax
from jax.experimental import pallas as pl
from jax.experimental.pallas import tpu as pltpu
from jax.experimental.pallas import tpu_sc as plsc

N = 10000
D = 128
E = 320000
NP = 10240
NC = 2
NS = 16
L = 16
NW = NC * NS
EB = 128
NBE = 80
EPT = NBE * EB
EPAD = NW * EPT
NBUF = 4
RPT = NP // NW
RC = 80
SEG = NP // NS

_MESH = plsc.VectorSubcoreMesh(core_axis_name="c", subcore_axis_name="s")


def _wid():
    return lax.axis_index("s") * NC + lax.axis_index("c")


def _hist_body(dst_hbm, out_hbm, idx_v, ones_v, zer_v, hist_sh, sem):
    c = lax.axis_index("c")
    s = lax.axis_index("s")

    def fill_zero(i, _):
        zer_v[pl.ds(i * L, L)] = jnp.zeros((L,), jnp.float32)
        return 0

    lax.fori_loop(0, SEG // L, fill_zero, 0)
    for i in range(EB // L):
        ones_v[pl.ds(i * L, L)] = jnp.ones((L,), jnp.float32)
    pltpu.sync_copy(zer_v, hist_sh.at[pl.ds(s * SEG, SEG)])
    plsc.subcore_barrier()

    pltpu.sync_copy(dst_hbm.at[_wid()], idx_v)

    def step(j, _):
        pltpu.sync_copy(ones_v, hist_sh.at[idx_v.at[j]], add=True)
        return 0

    lax.fori_loop(0, NBE, step, 0)
    plsc.subcore_barrier()
    pltpu.sync_copy(hist_sh.at[pl.ds(s * SEG, SEG)],
                    out_hbm.at[c, pl.ds(s * SEG, SEG)])


_hist = functools.partial(
    pl.kernel,
    out_type=jax.ShapeDtypeStruct((NC, NP), jnp.float32),
    mesh=_MESH,
    scratch_types=[
        pltpu.VMEM((NBE, EB), jnp.int32),
        pltpu.VMEM((EB,), jnp.float32),
        pltpu.VMEM((SEG,), jnp.float32),
        pltpu.VMEM_SHARED((NP,), jnp.float32),
        pltpu.SemaphoreType.DMA,
    ],
)(_hist_body)


DH = D // 2


def _agg_body(ya_hbm, yb_hbm, src_hbm, dst_hbm, out_hbm, si_v, di_v,
              *scratch):
    rows = scratch[:NBUF]
    zr_v = scratch[NBUF]
    acc_sh = scratch[NBUF + 1]
    gsem = scratch[NBUF + 2:NBUF + 2 + NBUF]
    ssem = scratch[NBUF + 2 + NBUF:]
    c = lax.axis_index("c")
    s = lax.axis_index("s")

    pltpu.sync_copy(src_hbm.at[_wid()], si_v)
    pltpu.sync_copy(dst_hbm.at[_wid()], di_v)

    def zero_row(i, _):
        for cc in range(DH // L):
            zr_v[i, pl.ds(cc * L, L)] = jnp.zeros((L,), jnp.float32)
        return 0

    lax.fori_loop(0, EB, zero_row, 0)

    for h, y_hbm in enumerate((ya_hbm, yb_hbm)):
        @pl.when(c == 0)
        def _():
            pltpu.sync_copy(y_hbm.at[pl.ds(s * SEG, SEG)],
                            acc_sh.at[pl.ds(s * SEG, SEG)])

        @pl.when(c != 0)
        def _():
            for k in range(SEG // EB):
                pltpu.sync_copy(zr_v, acc_sh.at[pl.ds(s * SEG + k * EB, EB)])

        plsc.subcore_barrier()

        for b in range(NBUF):
            pltpu.async_copy(y_hbm.at[si_v.at[b]], rows[b], gsem[b])

        def group(g, _):
            base = g * NBUF
            for b in range(NBUF):
                j = base + b
                pltpu.make_async_copy(y_hbm.at[si_v.at[j]], rows[b],
                                      gsem[b]).wait()
                pltpu.async_copy(rows[b], acc_sh.at[di_v.at[j]], ssem[b],
                                 add=True)
            for b in range(NBUF):
                j = base + b
                pltpu.make_async_copy(rows[b], acc_sh.at[di_v.at[j]],
                                      ssem[b]).wait()
                pltpu.async_copy(y_hbm.at[si_v.at[j + NBUF]], rows[b],
                                 gsem[b])
            return 0

        lax.fori_loop(0, NBE // NBUF - 1, group, 0)
        for b in range(NBUF):
            j = NBE - NBUF + b
            pltpu.make_async_copy(y_hbm.at[si_v.at[j]], rows[b],
                                  gsem[b]).wait()
            pltpu.async_copy(rows[b], acc_sh.at[di_v.at[j]], ssem[b],
                             add=True)
        for b in range(NBUF):
            j = NBE - NBUF + b
            pltpu.make_async_copy(rows[b], acc_sh.at[di_v.at[j]],
                                  ssem[b]).wait()
        plsc.subcore_barrier()
        @pl.when(c == 0)
        def _():
            pltpu.sync_copy(acc_sh.at[pl.ds(s * SEG, SEG)],
                            out_hbm.at[h, pl.ds(s * SEG, SEG), pl.ds(0, DH)])

        @pl.when(c != 0)
        def _():
            pltpu.sync_copy(acc_sh.at[pl.ds(s * SEG, SEG)],
                            out_hbm.at[h, pl.ds(s * SEG, SEG), pl.ds(DH, DH)])


_agg = functools.partial(
    pl.kernel,
    out_type=jax.ShapeDtypeStruct((2, NP, D), jnp.float32),
    mesh=_MESH,
    compiler_params=pltpu.CompilerParams(use_tc_tiling_on_sc=False),
    scratch_types=(
        [
            pltpu.VMEM((NBE, EB), jnp.int32),
            pltpu.VMEM((NBE, EB), jnp.int32),
        ]
        + [pltpu.VMEM((EB, DH), jnp.float32) for _ in range(NBUF)]
        + [
            pltpu.VMEM((EB, DH), jnp.float32),
            pltpu.VMEM_SHARED((NP, DH), jnp.float32),
        ]
        + [pltpu.SemaphoreType.DMA for _ in range(2 * NBUF)]
    ),
)(_agg_body)


_PREC = lax.Precision.HIGHEST
MB = 256


def _dot_t(a, b):
    return lax.dot_general(a, b, (((1,), (1,)), ((), ())),
                           preferred_element_type=jnp.float32,
                           precision=_PREC)


def _mm_body(x_ref, w_ref, o_ref):
    o_ref[...] = _dot_t(x_ref[...], w_ref[...])


def _mm(x, w):
    return pl.pallas_call(
        _mm_body,
        grid=(NP // MB,),
        in_specs=[
            pl.BlockSpec((MB, D), lambda i: (i, 0)),
            pl.BlockSpec((D, D), lambda i: (0, 0)),
        ],
        out_specs=pl.BlockSpec((MB, D), lambda i: (i, 0)),
        out_shape=jax.ShapeDtypeStruct((NP, D), jnp.float32),
    )(x, w)


def _dinv_body(h_ref, o_ref):
    o_ref[...] = lax.rsqrt(h_ref[0] + h_ref[1] + 1.0)


def _dinv(hist):
    return pl.pallas_call(
        _dinv_body,
        out_shape=jax.ShapeDtypeStruct((NP // D, D), jnp.float32),
    )(hist.reshape(NC, NP // D, D))


_P_SPECS = [
    pl.BlockSpec((1, MB, D), lambda i: (0, i, 0)),
    pl.BlockSpec((1, MB, D), lambda i: (1, i, 0)),
]
_H_SPEC = pl.BlockSpec((MB, DH), lambda i: (i, 0))


def _psum(p_ref):
    return p_ref[0, :, :DH] + p_ref[0, :, DH:]


def _k1_body(dv_ref, x_ref, w_ref, oa_ref, ob_ref):
    y1 = _dot_t(dv_ref[...] * x_ref[...], w_ref[...])
    oa_ref[...] = y1[:, :DH]
    ob_ref[...] = y1[:, DH:]


def _k1(dvb, x, w):
    return pl.pallas_call(
        _k1_body,
        grid=(NP // MB,),
        in_specs=[
            pl.BlockSpec((MB, D), lambda i: (i, 0)),
            pl.BlockSpec((MB, D), lambda i: (i, 0)),
            pl.BlockSpec((D, D), lambda i: (0, 0)),
        ],
        out_specs=[_H_SPEC, _H_SPEC],
        out_shape=[jax.ShapeDtypeStruct((NP, DH), jnp.float32)] * 2,
    )(dvb, x, w)


def _k2_body(dv_ref, pa, pb, b_ref, w_ref, oa_ref, ob_ref):
    dl, dr = dv_ref[:, :DH], dv_ref[:, DH:]
    tl = dl * _psum(pa) + b_ref[:, :DH]
    tr = dr * _psum(pb) + b_ref[:, DH:]
    hl = dl * jnp.maximum(tl, 0.0)
    hr = dr * jnp.maximum(tr, 0.0)
    y2 = _dot_t(hl, w_ref[:, :DH]) + _dot_t(hr, w_ref[:, DH:])
    oa_ref[...] = y2[:, :DH]
    ob_ref[...] = y2[:, DH:]


def _k2(dvb, p, b1, w2):
    return pl.pallas_call(
        _k2_body,
        grid=(NP // MB,),
        in_specs=[pl.BlockSpec((MB, D), lambda i: (i, 0))] + _P_SPECS + [
            pl.BlockSpec((1, D), lambda i: (0, 0)),
            pl.BlockSpec((D, D), lambda i: (0, 0)),
        ],
        out_specs=[_H_SPEC, _H_SPEC],
        out_shape=[jax.ShapeDtypeStruct((NP, DH), jnp.float32)] * 2,
    )(dvb, p, p, b1.reshape(1, D), w2)


def _k3_body(dv_ref, pa, pb, b_ref, o_ref):
    o_ref[:, :DH] = dv_ref[:, :DH] * _psum(pa) + b_ref[:, :DH]
    o_ref[:, DH:] = dv_ref[:, DH:] * _psum(pb) + b_ref[:, DH:]


def _k3(dvb, p, b2):
    return pl.pallas_call(
        _k3_body,
        grid=(NP // MB,),
        in_specs=[pl.BlockSpec((MB, D), lambda i: (i, 0))] + _P_SPECS + [
            pl.BlockSpec((1, D), lambda i: (0, 0)),
        ],
        out_specs=pl.BlockSpec((MB, D), lambda i: (i, 0)),
        out_shape=jax.ShapeDtypeStruct((NP, D), jnp.float32),
    )(dvb, p, p, b2.reshape(1, D))


_PAD_IDX = _np.int32(N) + _np.arange(EPAD - E, dtype=_np.int32) % (NP - N)


def kernel(x, edge_index, W1, b1, W2, b2):
    src = edge_index[0].astype(jnp.int32)
    dst = edge_index[1].astype(jnp.int32)
    pad = jnp.asarray(_PAD_IDX)
    src3 = jnp.concatenate([src, pad]).reshape(NW, NBE, EB)
    dst3 = jnp.concatenate([dst, pad]).reshape(NW, NBE, EB)
    x_pad = jnp.zeros((NP, D), jnp.float32).at[:N].set(x)

    hist = _hist(dst3)
    dvb = jnp.broadcast_to(
        _dinv(hist).reshape(NP)[:, None], (NP, D))
    ya, yb = _k1(dvb, x_pad, W1)
    p1 = _agg(ya, yb, src3, dst3)
    y2a, y2b = _k2(dvb, p1, b1, W2)
    p2 = _agg(y2a, y2b, src3, dst3)
    out = _k3(dvb, p2, b2)
    return out[:N]


# speedup vs baseline: 1.5343x; 1.0924x over previous
import functools

import numpy as _np

import jax
import jax.numpy as jnp
from jax import lax
from jax.experimental import pallas as pl
from jax.experimental.pallas import tpu as pltpu
from jax.experimental.pallas import tpu_sc as plsc

N = 10000
D = 128
E = 320000
NP = 10240
NC = 2
NS = 16
L = 16
NW = NC * NS
EB = 128
NBE = 80
EPT = NBE * EB
EPAD = NW * EPT
NBUF = 4
RPT = NP // NW
RC = 80
SEG = NP // NS

_MESH = plsc.VectorSubcoreMesh(core_axis_name="c", subcore_axis_name="s")


def _wid():
    return lax.axis_index("s") * NC + lax.axis_index("c")


def _hist_body(dst_hbm, out_hbm, idx_v, ones_v, zer_v, hist_sh, sem):
    c = lax.axis_index("c")
    s = lax.axis_index("s")

    def fill_zero(i, _):
        zer_v[pl.ds(i * L, L)] = jnp.zeros((L,), jnp.float32)
        return 0

    lax.fori_loop(0, SEG // L, fill_zero, 0)
    for i in range(EB // L):
        ones_v[pl.ds(i * L, L)] = jnp.ones((L,), jnp.float32)
    pltpu.sync_copy(zer_v, hist_sh.at[pl.ds(s * SEG, SEG)])
    plsc.subcore_barrier()

    pltpu.sync_copy(dst_hbm.at[_wid()], idx_v)

    def step(j, _):
        pltpu.sync_copy(ones_v, hist_sh.at[idx_v.at[j]], add=True)
        return 0

    lax.fori_loop(0, NBE, step, 0)
    plsc.subcore_barrier()
    pltpu.sync_copy(hist_sh.at[pl.ds(s * SEG, SEG)],
                    out_hbm.at[c, pl.ds(s * SEG, SEG)])


_hist = functools.partial(
    pl.kernel,
    out_type=jax.ShapeDtypeStruct((NC, NP), jnp.float32),
    mesh=_MESH,
    scratch_types=[
        pltpu.VMEM((NBE, EB), jnp.int32),
        pltpu.VMEM((EB,), jnp.float32),
        pltpu.VMEM((SEG,), jnp.float32),
        pltpu.VMEM_SHARED((NP,), jnp.float32),
        pltpu.SemaphoreType.DMA,
    ],
)(_hist_body)


DH = D // 2


def _agg_body(ya_hbm, yb_hbm, src_hbm, dst_hbm, out_hbm, si_v, di_v,
              *scratch):
    rows = scratch[:NBUF]
    zr_v = scratch[NBUF]
    acc_sh = scratch[NBUF + 1]
    gsem = scratch[NBUF + 2:NBUF + 2 + NBUF]
    ssem = scratch[NBUF + 2 + NBUF:]
    c = lax.axis_index("c")
    s = lax.axis_index("s")

    pltpu.sync_copy(src_hbm.at[_wid()], si_v)
    pltpu.sync_copy(dst_hbm.at[_wid()], di_v)

    def zero_row(i, _):
        for cc in range(DH // L):
            zr_v[i, pl.ds(cc * L, L)] = jnp.zeros((L,), jnp.float32)
        return 0

    lax.fori_loop(0, EB, zero_row, 0)

    for h, y_hbm in enumerate((ya_hbm, yb_hbm)):
        @pl.when(c == 0)
        def _():
            pltpu.sync_copy(y_hbm.at[pl.ds(s * SEG, SEG)],
                            acc_sh.at[pl.ds(s * SEG, SEG)])

        @pl.when(c != 0)
        def _():
            for k in range(SEG // EB):
                pltpu.sync_copy(zr_v, acc_sh.at[pl.ds(s * SEG + k * EB, EB)])

        plsc.subcore_barrier()

        for b in range(NBUF):
            pltpu.async_copy(y_hbm.at[si_v.at[b]], rows[b], gsem[b])

        def group(g, _):
            base = g * NBUF
            for b in range(NBUF):
                j = base + b
                pltpu.make_async_copy(y_hbm.at[si_v.at[j]], rows[b],
                                      gsem[b]).wait()
                pltpu.async_copy(rows[b], acc_sh.at[di_v.at[j]], ssem[b],
                                 add=True)
            for b in range(NBUF):
                j = base + b
                pltpu.make_async_copy(rows[b], acc_sh.at[di_v.at[j]],
                                      ssem[b]).wait()
                pltpu.async_copy(y_hbm.at[si_v.at[j + NBUF]], rows[b],
                                 gsem[b])
            return 0

        lax.fori_loop(0, NBE // NBUF - 1, group, 0)
        for b in range(NBUF):
            j = NBE - NBUF + b
            pltpu.make_async_copy(y_hbm.at[si_v.at[j]], rows[b],
                                  gsem[b]).wait()
            pltpu.async_copy(rows[b], acc_sh.at[di_v.at[j]], ssem[b],
                             add=True)
        for b in range(NBUF):
            j = NBE - NBUF + b
            pltpu.make_async_copy(rows[b], acc_sh.at[di_v.at[j]],
                                  ssem[b]).wait()
        plsc.subcore_barrier()
        @pl.when(c == 0)
        def _():
            pltpu.sync_copy(acc_sh.at[pl.ds(s * SEG, SEG)],
                            out_hbm.at[h, pl.ds(s * SEG, SEG), pl.ds(0, DH)])

        @pl.when(c != 0)
        def _():
            pltpu.sync_copy(acc_sh.at[pl.ds(s * SEG, SEG)],
                            out_hbm.at[h, pl.ds(s * SEG, SEG), pl.ds(DH, DH)])


_agg = functools.partial(
    pl.kernel,
    out_type=jax.ShapeDtypeStruct((2, NP, D), jnp.float32),
    mesh=_MESH,
    compiler_params=pltpu.CompilerParams(use_tc_tiling_on_sc=False),
    scratch_types=(
        [
            pltpu.VMEM((NBE, EB), jnp.int32),
            pltpu.VMEM((NBE, EB), jnp.int32),
        ]
        + [pltpu.VMEM((EB, DH), jnp.float32) for _ in range(NBUF)]
        + [
            pltpu.VMEM((EB, DH), jnp.float32),
            pltpu.VMEM_SHARED((NP, DH), jnp.float32),
        ]
        + [pltpu.SemaphoreType.DMA for _ in range(2 * NBUF)]
    ),
)(_agg_body)


_PREC = lax.Precision.HIGHEST
MB = 512


def _dot_t(a, b):
    return lax.dot_general(a, b, (((1,), (1,)), ((), ())),
                           preferred_element_type=jnp.float32,
                           precision=_PREC)


def _mm_body(x_ref, w_ref, o_ref):
    o_ref[...] = _dot_t(x_ref[...], w_ref[...])


def _mm(x, w):
    return pl.pallas_call(
        _mm_body,
        grid=(NP // MB,),
        in_specs=[
            pl.BlockSpec((MB, D), lambda i: (i, 0)),
            pl.BlockSpec((D, D), lambda i: (0, 0)),
        ],
        out_specs=pl.BlockSpec((MB, D), lambda i: (i, 0)),
        out_shape=jax.ShapeDtypeStruct((NP, D), jnp.float32),
    )(x, w)


def _dinv_body(h_ref, o_ref):
    o_ref[...] = lax.rsqrt(h_ref[0] + h_ref[1] + 1.0)


def _dinv(hist):
    return pl.pallas_call(
        _dinv_body,
        out_shape=jax.ShapeDtypeStruct((NP // D, D), jnp.float32),
    )(hist.reshape(NC, NP // D, D))


_P_SPECS = [
    pl.BlockSpec((1, MB, D), lambda i: (0, i, 0)),
    pl.BlockSpec((1, MB, D), lambda i: (1, i, 0)),
]
_H_SPEC = pl.BlockSpec((MB, DH), lambda i: (i, 0))


def _psum(p_ref):
    return p_ref[0, :, :DH] + p_ref[0, :, DH:]


def _k1_body(dv_ref, x_ref, w_ref, oa_ref, ob_ref):
    y1 = _dot_t(dv_ref[...] * x_ref[...], w_ref[...])
    oa_ref[...] = y1[:, :DH]
    ob_ref[...] = y1[:, DH:]


def _k1(dvb, x, w):
    return pl.pallas_call(
        _k1_body,
        grid=(NP // MB,),
        in_specs=[
            pl.BlockSpec((MB, D), lambda i: (i, 0)),
            pl.BlockSpec((MB, D), lambda i: (i, 0)),
            pl.BlockSpec((D, D), lambda i: (0, 0)),
        ],
        out_specs=[_H_SPEC, _H_SPEC],
        out_shape=[jax.ShapeDtypeStruct((NP, DH), jnp.float32)] * 2,
    )(dvb, x, w)


def _k2_body(dv_ref, pa, pb, b_ref, w_ref, oa_ref, ob_ref):
    dl, dr = dv_ref[:, :DH], dv_ref[:, DH:]
    tl = dl * _psum(pa) + b_ref[:, :DH]
    tr = dr * _psum(pb) + b_ref[:, DH:]
    hl = dl * jnp.maximum(tl, 0.0)
    hr = dr * jnp.maximum(tr, 0.0)
    y2 = _dot_t(hl, w_ref[:, :DH]) + _dot_t(hr, w_ref[:, DH:])
    oa_ref[...] = y2[:, :DH]
    ob_ref[...] = y2[:, DH:]


def _k2(dvb, p, b1, w2):
    return pl.pallas_call(
        _k2_body,
        grid=(NP // MB,),
        in_specs=[pl.BlockSpec((MB, D), lambda i: (i, 0))] + _P_SPECS + [
            pl.BlockSpec((1, D), lambda i: (0, 0)),
            pl.BlockSpec((D, D), lambda i: (0, 0)),
        ],
        out_specs=[_H_SPEC, _H_SPEC],
        out_shape=[jax.ShapeDtypeStruct((NP, DH), jnp.float32)] * 2,
    )(dvb, p, p, b1.reshape(1, D), w2)


def _k3_body(dv_ref, pa, pb, b_ref, o_ref):
    o_ref[:, :DH] = dv_ref[:, :DH] * _psum(pa) + b_ref[:, :DH]
    o_ref[:, DH:] = dv_ref[:, DH:] * _psum(pb) + b_ref[:, DH:]


def _k3(dvb, p, b2):
    return pl.pallas_call(
        _k3_body,
        grid=(40,),
        in_specs=[
            pl.BlockSpec((256, D), lambda i: (i, 0)),
            pl.BlockSpec((1, 256, D), lambda i: (0, i, 0)),
            pl.BlockSpec((1, 256, D), lambda i: (1, i, 0)),
            pl.BlockSpec((1, D), lambda i: (0, 0)),
        ],
        out_specs=pl.BlockSpec((256, D), lambda i: (i, 0)),
        out_shape=jax.ShapeDtypeStruct((N, D), jnp.float32),
    )(dvb, p, p, b2.reshape(1, D))


_PAD_IDX = _np.int32(N) + _np.arange(EPAD - E, dtype=_np.int32) % (NP - N)


def kernel(x, edge_index, W1, b1, W2, b2):
    ei = jnp.concatenate(
        [edge_index.astype(jnp.int32),
         jnp.broadcast_to(jnp.asarray(_PAD_IDX), (2, EPAD - E))], axis=1)
    src3 = ei[0].reshape(NW, NBE, EB)
    dst3 = ei[1].reshape(NW, NBE, EB)
    x_pad = jnp.zeros((NP, D), jnp.float32).at[:N].set(x)

    hist = _hist(dst3)
    dvb = jnp.broadcast_to(
        _dinv(hist).reshape(NP)[:, None], (NP, D))
    ya, yb = _k1(dvb, x_pad, W1)
    p1 = _agg(ya, yb, src3, dst3)
    y2a, y2b = _k2(dvb, p1, b1, W2)
    p2 = _agg(y2a, y2b, src3, dst3)
    return _k3(dvb, p2, b2)


# speedup vs baseline: 1.5348x; 1.0003x over previous
import functools

import numpy as _np

import jax
import jax.numpy as jnp
from jax import lax
from jax.experimental import pallas as pl
from jax.experimental.pallas import tpu as pltpu
from jax.experimental.pallas import tpu_sc as plsc

N = 10000
D = 128
E = 320000
NP = 10240
NC = 2
NS = 16
L = 16
NW = NC * NS
EB = 128
NBE = 80
EPT = NBE * EB
EPAD = NW * EPT
NBUF = 4
SEG = NP // NS

_MESH = plsc.VectorSubcoreMesh(core_axis_name="c", subcore_axis_name="s")


def _wid():
    return lax.axis_index("s") * NC + lax.axis_index("c")


def _hist_body(dst_hbm, out_hbm, idx_v, ones_v, zer_v, hist_sh, sem):
    c = lax.axis_index("c")
    s = lax.axis_index("s")

    def fill_zero(i, _):
        zer_v[pl.ds(i * L, L)] = jnp.zeros((L,), jnp.float32)
        return 0

    lax.fori_loop(0, SEG // L, fill_zero, 0)
    for i in range(EB // L):
        ones_v[pl.ds(i * L, L)] = jnp.ones((L,), jnp.float32)
    pltpu.sync_copy(zer_v, hist_sh.at[pl.ds(s * SEG, SEG)])
    plsc.subcore_barrier()

    pltpu.sync_copy(dst_hbm.at[_wid()], idx_v)

    def step(j, _):
        pltpu.sync_copy(ones_v, hist_sh.at[idx_v.at[j]], add=True)
        return 0

    lax.fori_loop(0, NBE, step, 0)
    plsc.subcore_barrier()
    pltpu.sync_copy(hist_sh.at[pl.ds(s * SEG, SEG)],
                    out_hbm.at[c, pl.ds(s * SEG, SEG)])


_hist = functools.partial(
    pl.kernel,
    out_type=jax.ShapeDtypeStruct((NC, NP), jnp.float32),
    mesh=_MESH,
    scratch_types=[
        pltpu.VMEM((NBE, EB), jnp.int32),
        pltpu.VMEM((EB,), jnp.float32),
        pltpu.VMEM((SEG,), jnp.float32),
        pltpu.VMEM_SHARED((NP,), jnp.float32),
        pltpu.SemaphoreType.DMA,
    ],
)(_hist_body)


DH = D // 2


def _agg_body(ya_hbm, yb_hbm, src_hbm, dst_hbm, out_hbm, si_v, di_v,
              *scratch):
    rows = scratch[:NBUF]
    zr_v = scratch[NBUF]
    acc_sh = scratch[NBUF + 1]
    gsem = scratch[NBUF + 2:NBUF + 2 + NBUF]
    ssem = scratch[NBUF + 2 + NBUF:]
    c = lax.axis_index("c")
    s = lax.axis_index("s")

    pltpu.sync_copy(src_hbm.at[_wid()], si_v)
    pltpu.sync_copy(dst_hbm.at[_wid()], di_v)

    def zero_row(i, _):
        for cc in range(DH // L):
            zr_v[i, pl.ds(cc * L, L)] = jnp.zeros((L,), jnp.float32)
        return 0

    lax.fori_loop(0, EB, zero_row, 0)

    for h, y_hbm in enumerate((ya_hbm, yb_hbm)):
        @pl.when(c == 0)
        def _():
            pltpu.sync_copy(y_hbm.at[pl.ds(s * SEG, SEG)],
                            acc_sh.at[pl.ds(s * SEG, SEG)])

        @pl.when(c != 0)
        def _():
            for k in range(SEG // EB):
                pltpu.sync_copy(zr_v, acc_sh.at[pl.ds(s * SEG + k * EB, EB)])

        plsc.subcore_barrier()

        for b in range(NBUF):
            pltpu.async_copy(y_hbm.at[si_v.at[b]], rows[b], gsem[b])

        def group(g, _):
            base = g * NBUF
            for b in range(NBUF):
                j = base + b
                pltpu.make_async_copy(y_hbm.at[si_v.at[j]], rows[b],
                                      gsem[b]).wait()
                pltpu.async_copy(rows[b], acc_sh.at[di_v.at[j]], ssem[b],
                                 add=True)
            for b in range(NBUF):
                j = base + b
                pltpu.make_async_copy(rows[b], acc_sh.at[di_v.at[j]],
                                      ssem[b]).wait()
                pltpu.async_copy(y_hbm.at[si_v.at[j + NBUF]], rows[b],
                                 gsem[b])
            return 0

        lax.fori_loop(0, NBE // NBUF - 1, group, 0)
        for b in range(NBUF):
            j = NBE - NBUF + b
            pltpu.make_async_copy(y_hbm.at[si_v.at[j]], rows[b],
                                  gsem[b]).wait()
            pltpu.async_copy(rows[b], acc_sh.at[di_v.at[j]], ssem[b],
                             add=True)
        for b in range(NBUF):
            j = NBE - NBUF + b
            pltpu.make_async_copy(rows[b], acc_sh.at[di_v.at[j]],
                                  ssem[b]).wait()
        plsc.subcore_barrier()
        @pl.when(c == 0)
        def _():
            pltpu.sync_copy(acc_sh.at[pl.ds(s * SEG, SEG)],
                            out_hbm.at[h, pl.ds(s * SEG, SEG), pl.ds(0, DH)])

        @pl.when(c != 0)
        def _():
            pltpu.sync_copy(acc_sh.at[pl.ds(s * SEG, SEG)],
                            out_hbm.at[h, pl.ds(s * SEG, SEG), pl.ds(DH, DH)])


_agg = functools.partial(
    pl.kernel,
    out_type=jax.ShapeDtypeStruct((2, NP, D), jnp.float32),
    mesh=_MESH,
    compiler_params=pltpu.CompilerParams(use_tc_tiling_on_sc=False),
    scratch_types=(
        [
            pltpu.VMEM((NBE, EB), jnp.int32),
            pltpu.VMEM((NBE, EB), jnp.int32),
        ]
        + [pltpu.VMEM((EB, DH), jnp.float32) for _ in range(NBUF)]
        + [
            pltpu.VMEM((EB, DH), jnp.float32),
            pltpu.VMEM_SHARED((NP, DH), jnp.float32),
        ]
        + [pltpu.SemaphoreType.DMA for _ in range(2 * NBUF)]
    ),
)(_agg_body)


_PREC = lax.Precision.HIGHEST
MB = 512


def _dot_t(a, b):
    return lax.dot_general(a, b, (((1,), (1,)), ((), ())),
                           preferred_element_type=jnp.float32,
                           precision=_PREC)


def _mm_body(x_ref, w_ref, o_ref):
    o_ref[...] = _dot_t(x_ref[...], w_ref[...])


def _mm(x, w):
    return pl.pallas_call(
        _mm_body,
        grid=(NP // MB,),
        in_specs=[
            pl.BlockSpec((MB, D), lambda i: (i, 0)),
            pl.BlockSpec((D, D), lambda i: (0, 0)),
        ],
        out_specs=pl.BlockSpec((MB, D), lambda i: (i, 0)),
        out_shape=jax.ShapeDtypeStruct((NP, D), jnp.float32),
    )(x, w)


def _dinv_body(h_ref, o_ref):
    o_ref[...] = lax.rsqrt(h_ref[0] + h_ref[1] + 1.0)


def _dinv(hist):
    return pl.pallas_call(
        _dinv_body,
        out_shape=jax.ShapeDtypeStruct((NP // D, D), jnp.float32),
    )(hist.reshape(NC, NP // D, D))


_P_SPECS = [
    pl.BlockSpec((1, MB, D), lambda i: (0, i, 0)),
    pl.BlockSpec((1, MB, D), lambda i: (1, i, 0)),
]
_H_SPEC = pl.BlockSpec((MB, DH), lambda i: (i, 0))


def _psum(p_ref):
    return p_ref[0, :, :DH] + p_ref[0, :, DH:]


def _k1_body(dv_ref, x_ref, w_ref, oa_ref, ob_ref):
    y1 = _dot_t(dv_ref[...] * x_ref[...], w_ref[...])
    oa_ref[...] = y1[:, :DH]
    ob_ref[...] = y1[:, DH:]


def _k1(dvb, x, w):
    return pl.pallas_call(
        _k1_body,
        grid=(NP // MB,),
        in_specs=[
            pl.BlockSpec((MB, D), lambda i: (i, 0)),
            pl.BlockSpec((MB, D), lambda i: (i, 0)),
            pl.BlockSpec((D, D), lambda i: (0, 0)),
        ],
        out_specs=[_H_SPEC, _H_SPEC],
        out_shape=[jax.ShapeDtypeStruct((NP, DH), jnp.float32)] * 2,
    )(dvb, x, w)


def _k2_body(dv_ref, pa, pb, b_ref, w_ref, oa_ref, ob_ref):
    dl, dr = dv_ref[:, :DH], dv_ref[:, DH:]
    tl = dl * _psum(pa) + b_ref[:, :DH]
    tr = dr * _psum(pb) + b_ref[:, DH:]
    hl = dl * jnp.maximum(tl, 0.0)
    hr = dr * jnp.maximum(tr, 0.0)
    y2 = _dot_t(hl, w_ref[:, :DH]) + _dot_t(hr, w_ref[:, DH:])
    oa_ref[...] = y2[:, :DH]
    ob_ref[...] = y2[:, DH:]


def _k2(dvb, p, b1, w2):
    return pl.pallas_call(
        _k2_body,
        grid=(NP // MB,),
        in_specs=[pl.BlockSpec((MB, D), lambda i: (i, 0))] + _P_SPECS + [
            pl.BlockSpec((1, D), lambda i: (0, 0)),
            pl.BlockSpec((D, D), lambda i: (0, 0)),
        ],
        out_specs=[_H_SPEC, _H_SPEC],
        out_shape=[jax.ShapeDtypeStruct((NP, DH), jnp.float32)] * 2,
    )(dvb, p, p, b1.reshape(1, D), w2)


def _k3_body(dv_ref, pa, pb, b_ref, o_ref):
    o_ref[:, :DH] = dv_ref[:, :DH] * _psum(pa) + b_ref[:, :DH]
    o_ref[:, DH:] = dv_ref[:, DH:] * _psum(pb) + b_ref[:, DH:]


def _k3(dvb, p, b2):
    return pl.pallas_call(
        _k3_body,
        grid=(40,),
        in_specs=[
            pl.BlockSpec((256, D), lambda i: (i, 0)),
            pl.BlockSpec((1, 256, D), lambda i: (0, i, 0)),
            pl.BlockSpec((1, 256, D), lambda i: (1, i, 0)),
            pl.BlockSpec((1, D), lambda i: (0, 0)),
        ],
        out_specs=pl.BlockSpec((256, D), lambda i: (i, 0)),
        out_shape=jax.ShapeDtypeStruct((N, D), jnp.float32),
    )(dvb, p, p, b2.reshape(1, D))


_PAD_IDX = _np.int32(N) + _np.arange(EPAD - E, dtype=_np.int32) % (NP - N)


def kernel(x, edge_index, W1, b1, W2, b2):
    ei = jnp.concatenate(
        [edge_index.astype(jnp.int32),
         jnp.broadcast_to(jnp.asarray(_PAD_IDX), (2, EPAD - E))], axis=1)
    src3 = ei[0].reshape(NW, NBE, EB)
    dst3 = ei[1].reshape(NW, NBE, EB)
    x_pad = jnp.zeros((NP, D), jnp.float32).at[:N].set(x)

    hist = _hist(dst3)
    dvb = jnp.broadcast_to(
        _dinv(hist).reshape(NP)[:, None], (NP, D))
    ya, yb = _k1(dvb, x_pad, W1)
    p1 = _agg(ya, yb, src3, dst3)
    y2a, y2b = _k2(dvb, p1, b1, W2)
    p2 = _agg(y2a, y2b, src3, dst3)
    return _k3(dvb, p2, b2)
